# baseline (speedup 1.0000x reference)
import functools

import jax
import jax.numpy as jnp
from jax import lax
from jax.experimental import pallas as pl
from jax.experimental.pallas import tpu as pltpu
from jax.experimental.pallas import tpu_sc as plsc

VOCAB = 1_000_000
DIM = 64
B = 4096 * 50
NW = 32
B_PER_W = B // NW
CHUNK = 128
N_CHUNKS = B_PER_W // CHUNK

CW = 256
N_FULL = VOCAB // CW
TAIL_W = VOCAB - N_FULL * CW
TOTAL_CH = N_FULL + 1
MAX_ITERS = (TOTAL_CH + NW - 1) // NW

_mesh = plsc.VectorSubcoreMesh(core_axis_name="c", subcore_axis_name="s")


@functools.partial(
    pl.kernel,
    mesh=_mesh,
    out_type=jax.ShapeDtypeStruct((VOCAB * DIM,), jnp.float32),
    scratch_types=[
        pltpu.VMEM((DIM, CW), jnp.float32),
        pltpu.VMEM((CW * DIM,), jnp.float32),
        pltpu.VMEM((TAIL_W * DIM,), jnp.float32),
    ],
    compiler_params=pltpu.CompilerParams(
        use_tc_tiling_on_sc=True, needs_layout_passes=False
    ),
)
def _sc_detile(wt_hbm, tail_hbm, out_hbm, inbuf, outbuf, tailbuf):
    wid = lax.axis_index("s") * 2 + lax.axis_index("c")
    i64 = lax.iota(jnp.int32, 16) * DIM

    @pl.when(wid == NW - 1)
    def _():
        pltpu.sync_copy(tail_hbm, tailbuf)
        pltpu.sync_copy(
            tailbuf, out_hbm.at[pl.ds(N_FULL * (CW * DIM), TAIL_W * DIM)]
        )

    def chunk_body(i, _):
        c = wid + i * NW

        @pl.when(c < N_FULL)
        def _():
            v0 = pl.multiple_of(c * CW, CW)
            pltpu.sync_copy(wt_hbm.at[:, pl.ds(v0, CW)], inbuf)

            def f_body(f, _):
                fv = i64 + f
                for t in range(CW // 128):
                    for vv in range(8):
                        vec = inbuf[f, pl.ds(t * 128 + vv * 16, 16)]
                        idx = fv + (t * 8192 + vv * 1024)
                        plsc.store_scatter(outbuf, [idx], vec)
                return _

            lax.fori_loop(0, DIM, f_body, None)
            pltpu.sync_copy(outbuf, out_hbm.at[pl.ds(c * (CW * DIM), CW * DIM)])

        return _

    lax.fori_loop(0, MAX_ITERS, chunk_body, None)


@functools.partial(
    pl.kernel,
    mesh=_mesh,
    out_type=jax.ShapeDtypeStruct((B, DIM), jnp.float32),
    scratch_types=[
        pltpu.VMEM((N_CHUNKS, CHUNK), jnp.int32),
        pltpu.VMEM((2, CHUNK, DIM), jnp.float32),
        pltpu.SemaphoreType.DMA,
        pltpu.SemaphoreType.DMA,
    ],
    compiler_params=pltpu.CompilerParams(use_tc_tiling_on_sc=False),
)
def _sc_gather(idx_hbm, table_hbm, out_hbm, idx_v, rows_v, gsem, ssem):
    wid = lax.axis_index("s") * 2 + lax.axis_index("c")
    row0 = wid * N_CHUNKS

    pltpu.sync_copy(idx_hbm.at[wid], idx_v)

    pltpu.async_copy(table_hbm.at[idx_v.at[0]], rows_v.at[0], gsem)

    def step(j, _):
        buf = lax.rem(j, 2)
        nbuf = 1 - buf
        @pl.when(j + 1 < N_CHUNKS)
        def _():
            pltpu.async_copy(
                table_hbm.at[idx_v.at[j + 1]], rows_v.at[nbuf], gsem
            )

        pltpu.make_async_copy(
            table_hbm.at[idx_v.at[j]], rows_v.at[buf], gsem
        ).wait()
        out_slice = out_hbm.at[pl.ds((row0 + j) * CHUNK, CHUNK), :]
        pltpu.async_copy(rows_v.at[buf], out_slice, ssem)
        @pl.when(j >= 1)
        def _():
            prev = lax.rem(j + 1, 2)
            prev_slice = out_hbm.at[pl.ds((row0 + j - 1) * CHUNK, CHUNK), :]
            pltpu.make_async_copy(rows_v.at[prev], prev_slice, ssem).wait()
        return _

    lax.fori_loop(0, N_CHUNKS, step, None)
    last = N_CHUNKS - 1
    pltpu.make_async_copy(
        rows_v.at[lax.rem(last, 2)],
        out_hbm.at[pl.ds((row0 + last) * CHUNK, CHUNK), :],
        ssem,
    ).wait()


def kernel(token_ids, weight):
    wt = weight.T
    tail = weight[N_FULL * CW :].reshape(TAIL_W * DIM)
    w1d = _sc_detile(wt, tail)
    table = w1d.reshape(VOCAB, DIM)
    idx = token_ids.reshape(NW, N_CHUNKS, CHUNK).astype(jnp.int32)
    out = _sc_gather(idx, table)
    return out.reshape(token_ids.shape + (DIM,))
